# Initial kernel scaffold; baseline (speedup 1.0000x reference)
import jax
import jax.numpy as jnp
from jax.experimental import pallas as pl


def kernel(x, lap_rows, lap_cols, lap_vals, weight, bias):
    raise NotImplementedError("write your pallas kernel here")


# speedup vs baseline: 3.1024x; 3.1024x over previous
import functools

import jax
import jax.numpy as jnp
from jax import lax
from jax.experimental import pallas as pl
from jax.experimental.pallas import tpu as pltpu
from jax.experimental.pallas import tpu_sc as plsc

_VPAD = 10240
_CH = 128
_NS = 16
_NC = 2
_WB = 64


def _make_spmm(n_half, alpha, beta, ept):
    mesh = plsc.VectorSubcoreMesh(core_axis_name="c", subcore_axis_name="s")
    eps = ept // _NS
    nchunk = eps // _CH
    rps = _VPAD // _NS
    nwb = rps // _WB
    nj = n_half // 16

    def body(xc, xp, rows, cols, vals, y, acc, rowsv, colsv, valsv, gbuf,
             abuf, pbuf, sem):
        c = lax.axis_index("c")
        s = lax.axis_index("s")
        row0 = s * rps
        coff = c * _VPAD

        def zrow(e, carry):
            for j in range(nj):
                abuf[e, pl.ds(j * 16, 16)] = jnp.zeros((16,), jnp.float32)
            return carry
        lax.fori_loop(0, _WB, zrow, 0)
        for k in range(nwb):
            pltpu.sync_copy(abuf, acc.at[pl.ds(row0 + k * _WB, _WB)])
        plsc.subcore_barrier()

        ebase = s * eps

        def echunk(i, carry):
            e0 = ebase + i * _CH
            pltpu.sync_copy(rows.at[pl.ds(e0, _CH)], rowsv)
            pltpu.sync_copy(cols.at[pl.ds(e0, _CH)], colsv)
            pltpu.sync_copy(vals.at[pl.ds(e0, _CH)], valsv)
            for j in range(_CH // 16):
                colsv[pl.ds(j * 16, 16)] = colsv[pl.ds(j * 16, 16)] + coff
            pltpu.async_copy(xc.at[colsv], gbuf, sem).wait()

            def scale(g, cc):
                vv = valsv[pl.ds(g * 16, 16)]
                for t in range(16):
                    e = g * 16 + t
                    vb = vv[t]
                    for j in range(nj):
                        gbuf[e, pl.ds(j * 16, 16)] = (
                            gbuf[e, pl.ds(j * 16, 16)] * vb)
                return cc
            lax.fori_loop(0, _CH // 16, scale, 0)
            pltpu.sync_copy(gbuf, acc.at[rowsv], add=True)
            return carry
        lax.fori_loop(0, nchunk, echunk, 0)
        plsc.subcore_barrier()

        for k in range(nwb):
            r0 = row0 + k * _WB
            pltpu.sync_copy(acc.at[pl.ds(r0, _WB)], abuf)
            if beta != 0.0:
                pltpu.sync_copy(xp.at[pl.ds(coff + r0, _WB)], pbuf)
            if alpha != 1.0 or beta != 0.0:
                def wrow(e, cc):
                    for j in range(nj):
                        g = abuf[e, pl.ds(j * 16, 16)] * alpha
                        if beta != 0.0:
                            g = g + pbuf[e, pl.ds(j * 16, 16)] * beta
                        abuf[e, pl.ds(j * 16, 16)] = g
                    return cc
                lax.fori_loop(0, _WB, wrow, 0)
            pltpu.sync_copy(abuf, y.at[pl.ds(coff + r0, _WB)])

    return pl.kernel(
        body,
        out_type=jax.ShapeDtypeStruct((_NC * _VPAD, n_half), jnp.float32),
        mesh=mesh,
        scratch_types=[
            pltpu.VMEM_SHARED((_VPAD, n_half), jnp.float32),
            pltpu.VMEM((_CH,), jnp.int32),
            pltpu.VMEM((_CH,), jnp.int32),
            pltpu.VMEM((_CH,), jnp.float32),
            pltpu.VMEM((_CH, n_half), jnp.float32),
            pltpu.VMEM((_WB, n_half), jnp.float32),
            pltpu.VMEM((_WB, n_half), jnp.float32),
            pltpu.SemaphoreType.DMA,
        ],
    )


def _tensordot_tc(xks, weight, bias, v_out):
    r_num = weight.shape[0]
    cin = weight.shape[1]
    cout = weight.shape[2]
    bsz = xks[0].shape[0]
    blk = 1280
    nb = _VPAD // blk

    def tc_body(*refs):
        xrefs = refs[:r_num]
        w_ref = refs[r_num]
        b_ref = refs[r_num + 1]
        o_ref = refs[r_num + 2]
        for b in range(bsz):
            acc = jnp.zeros((cout, blk), jnp.float32)
            for r in range(r_num):
                acc = acc + lax.dot_general(
                    w_ref[r], xrefs[r][b],
                    (((0,), (1,)), ((), ())),
                    preferred_element_type=jnp.float32)
            o_ref[b] = acc + b_ref[0][:, None]

    in_specs = [pl.BlockSpec((bsz, blk, cin), lambda i: (0, i, 0))
                for _ in range(r_num)]
    in_specs.append(pl.BlockSpec((r_num, cin, cout), lambda i: (0, 0, 0)))
    in_specs.append(pl.BlockSpec((1, cout), lambda i: (0, 0)))
    out = pl.pallas_call(
        tc_body,
        grid=(nb,),
        in_specs=in_specs,
        out_specs=pl.BlockSpec((bsz, cout, blk), lambda i: (0, 0, i)),
        out_shape=jax.ShapeDtypeStruct((bsz, cout, _VPAD), jnp.float32),
    )(*xks, weight, bias.reshape(1, cout))
    return out[:, :, :v_out]


def kernel(x, lap_rows, lap_cols, lap_vals, weight, bias):
    bsz, cin, v_num = x.shape
    r_num = weight.shape[0]
    rows = lap_rows.astype(jnp.int32)
    cols = lap_cols.astype(jnp.int32)
    vals = lap_vals.astype(jnp.float32)
    e2 = rows.shape[0]
    eps = ((e2 + _NS * _CH - 1) // (_NS * _CH)) * _CH
    ept = eps * _NS
    padn = ept - e2
    rows = jnp.concatenate([rows, jnp.full((padn,), _VPAD - 1, jnp.int32)])
    cols = jnp.concatenate([cols, jnp.zeros((padn,), jnp.int32)])
    vals = jnp.concatenate([vals, jnp.zeros((padn,), jnp.float32)])

    xt = jnp.transpose(x, (0, 2, 1))
    x0 = (jnp.zeros((bsz, _VPAD, cin), jnp.float32)
          .at[:, :v_num, :].set(xt)
          .reshape(bsz * _VPAD, cin))

    spmm_first = _make_spmm(cin, 1.0, 0.0, ept)
    spmm_cheb = _make_spmm(cin, 2.0, -1.0, ept)

    xs = [x0]
    if r_num > 1:
        xs.append(spmm_first(x0, x0, rows, cols, vals))
        for _ in range(2, r_num):
            xs.append(spmm_cheb(xs[-1], xs[-2], rows, cols, vals))

    xks = [a.reshape(bsz, _VPAD, cin) for a in xs]
    return _tensordot_tc(xks, weight, bias, v_num)
